# baseline (speedup 1.0000x reference)
import functools

import jax
import jax.numpy as jnp
from jax.experimental import pallas as pl
from jax.experimental.pallas import tpu as pltpu

E = 8
K = 2
D = 1024
H = 2048
T = 2048
HB = 512
NH = H // HB
EPAD = 128


def _erf(x):
    a1, a2, a3, a4, a5 = (0.254829592, -0.284496736, 1.421413741,
                          -1.453152027, 1.061405429)
    p = 0.3275911
    ax = jnp.abs(x)
    t = 1.0 / (1.0 + p * ax)
    poly = t * (a1 + t * (a2 + t * (a3 + t * (a4 + t * a5))))
    y = 1.0 - poly * jnp.exp(-ax * ax)
    return jnp.sign(x) * y


def _gelu(x):
    return 0.5 * x * (1.0 + _erf(x * 0.7071067811865476))


def _moe_body(x_ref, xb_ref, wg_ref, bg_ref, w1_ref, b1_ref, w2_ref, b2_ref,
              out_ref, wall_ref):
    e = pl.program_id(0)
    h = pl.program_id(1)

    @pl.when((e == 0) & (h == 0))
    def _gate():
        logits = jnp.dot(x_ref[...], wg_ref[...],
                         preferred_element_type=jnp.float32) + bg_ref[...]
        lane = jax.lax.broadcasted_iota(jnp.int32, (T, EPAD), 1)
        m1 = jnp.max(logits, axis=1, keepdims=True)
        a1 = jnp.min(jnp.where(logits == m1, lane, EPAD), axis=1,
                     keepdims=True)
        l2 = jnp.where(lane == a1, -3e38, logits)
        m2 = jnp.max(l2, axis=1, keepdims=True)
        a2 = jnp.min(jnp.where(l2 == m2, lane, EPAD), axis=1, keepdims=True)
        g2 = 1.0 / (1.0 + jnp.exp(m1 - m2))
        g1 = 1.0 - g2
        wall_ref[...] = (jnp.where(lane == a1, g1, 0.0) +
                         jnp.where(lane == a2, g2, 0.0))

    lane = jax.lax.broadcasted_iota(jnp.int32, (T, EPAD), 1)
    w_col = jnp.sum(jnp.where(lane == e, wall_ref[...], 0.0), axis=1,
                    keepdims=True)

    hidden = _gelu(jnp.dot(xb_ref[...], w1_ref[0],
                           preferred_element_type=jnp.float32) + b1_ref[0])
    part = jnp.dot(hidden.astype(jnp.bfloat16), w2_ref[0],
                   preferred_element_type=jnp.float32)
    part = jnp.where(h == 0, part + b2_ref[0], part)
    contrib = w_col * part

    @pl.when((e == 0) & (h == 0))
    def _init():
        out_ref[...] = contrib

    @pl.when((e > 0) | (h > 0))
    def _acc():
        out_ref[...] += contrib


@jax.jit
def _moe(x, xb, wg_pad, bg_pad, w1, b1, w2, b2):
    return pl.pallas_call(
        _moe_body,
        grid=(E, NH),
        in_specs=[
            pl.BlockSpec((T, D), lambda e, h: (0, 0)),
            pl.BlockSpec((T, D), lambda e, h: (0, 0)),
            pl.BlockSpec((D, EPAD), lambda e, h: (0, 0)),
            pl.BlockSpec((1, EPAD), lambda e, h: (0, 0)),
            pl.BlockSpec((1, D, HB), lambda e, h: (e, 0, h)),
            pl.BlockSpec((1, 1, HB), lambda e, h: (e, 0, h)),
            pl.BlockSpec((1, HB, D), lambda e, h: (e, h, 0)),
            pl.BlockSpec((1, 1, D), lambda e, h: (e, 0, 0)),
        ],
        out_specs=pl.BlockSpec((T, D), lambda e, h: (0, 0)),
        out_shape=jax.ShapeDtypeStruct((T, D), jnp.float32),
        scratch_shapes=[pltpu.VMEM((T, EPAD), jnp.float32)],
    )(x, xb, wg_pad, bg_pad, w1, b1, w2, b2)


def kernel(hidden_states, Wg, bg, W1, b1, W2, b2, bias):
    orig_shape = hidden_states.shape
    x = hidden_states.reshape(-1, orig_shape[-1])
    xb = x.astype(jnp.bfloat16)
    wg_pad = jnp.zeros((D, EPAD), jnp.float32).at[:, :E].set(Wg)
    bg_pad = jnp.full((1, EPAD), -3e38, jnp.float32).at[0, :E].set(bg)
    out = _moe(x, xb, wg_pad, bg_pad,
               W1.astype(jnp.bfloat16), b1.reshape(E, 1, H),
               W2.astype(jnp.bfloat16), b2.reshape(E, 1, D))
    return (out.reshape(orig_shape), bias)


# speedup vs baseline: 1.4225x; 1.4225x over previous
import functools

import jax
import jax.numpy as jnp
from jax.experimental import pallas as pl
from jax.experimental.pallas import tpu as pltpu

E = 8
K = 2
D = 1024
H = 2048
T = 2048
HB = 1024
NH = H // HB
EPAD = 128


def _gelu(x):
    u = 0.7978845608028654 * (x + 0.044715 * x * x * x)
    return 0.5 * x * (1.0 + jnp.tanh(u))


def _moe_body(x_ref, xb_ref, wg_ref, bg_ref, b2all_ref,
              w1_ref, b1_ref, w2_ref, out_ref, wall_ref):
    e = pl.program_id(0)
    h = pl.program_id(1)

    @pl.when((e == 0) & (h == 0))
    def _gate():
        logits = jnp.dot(x_ref[...], wg_ref[...],
                         preferred_element_type=jnp.float32) + bg_ref[...]
        lane = jax.lax.broadcasted_iota(jnp.int32, (T, EPAD), 1)
        m1 = jnp.max(logits, axis=1, keepdims=True)
        a1 = jnp.min(jnp.where(logits == m1, lane, EPAD), axis=1,
                     keepdims=True)
        l2 = jnp.where(lane == a1, -3e38, logits)
        m2 = jnp.max(l2, axis=1, keepdims=True)
        a2 = jnp.min(jnp.where(l2 == m2, lane, EPAD), axis=1, keepdims=True)
        g2 = 1.0 / (1.0 + jnp.exp(m1 - m2))
        g1 = 1.0 - g2
        wall = (jnp.where(lane == a1, g1, 0.0) +
                jnp.where(lane == a2, g2, 0.0))
        wall_ref[...] = wall
        out_ref[...] = jnp.dot(wall, b2all_ref[...],
                               preferred_element_type=jnp.float32)

    lane = jax.lax.broadcasted_iota(jnp.int32, (T, EPAD), 1)
    w_col = jnp.sum(jnp.where(lane == e, wall_ref[...], 0.0), axis=1,
                    keepdims=True)

    hidden = _gelu(jnp.dot(xb_ref[...], w1_ref[0],
                           preferred_element_type=jnp.float32) + b1_ref[0])
    hb = (w_col * hidden).astype(jnp.bfloat16)
    out_ref[...] += jnp.dot(hb, w2_ref[0],
                            preferred_element_type=jnp.float32)


@jax.jit
def _moe(x, xb, wg_pad, bg_pad, b2all, w1, b1, w2):
    return pl.pallas_call(
        _moe_body,
        grid=(E, NH),
        in_specs=[
            pl.BlockSpec((T, D), lambda e, h: (0, 0)),
            pl.BlockSpec((T, D), lambda e, h: (0, 0)),
            pl.BlockSpec((D, EPAD), lambda e, h: (0, 0)),
            pl.BlockSpec((1, EPAD), lambda e, h: (0, 0)),
            pl.BlockSpec((EPAD, D), lambda e, h: (0, 0)),
            pl.BlockSpec((1, D, HB), lambda e, h: (e, 0, h)),
            pl.BlockSpec((1, 1, HB), lambda e, h: (e, 0, h)),
            pl.BlockSpec((1, HB, D), lambda e, h: (e, h, 0)),
        ],
        out_specs=pl.BlockSpec((T, D), lambda e, h: (0, 0)),
        out_shape=jax.ShapeDtypeStruct((T, D), jnp.float32),
        scratch_shapes=[pltpu.VMEM((T, EPAD), jnp.float32)],
    )(x, xb, wg_pad, bg_pad, b2all, w1, b1, w2)


def kernel(hidden_states, Wg, bg, W1, b1, W2, b2, bias):
    orig_shape = hidden_states.shape
    x = hidden_states.reshape(-1, orig_shape[-1])
    xb = x.astype(jnp.bfloat16)
    wg_pad = jnp.zeros((D, EPAD), jnp.float32).at[:, :E].set(Wg)
    bg_pad = jnp.full((1, EPAD), -3e38, jnp.float32).at[0, :E].set(bg)
    b2all = jnp.zeros((EPAD, D), jnp.float32).at[:E].set(b2)
    out = _moe(x, xb, wg_pad, bg_pad, b2all,
               W1.astype(jnp.bfloat16), b1.reshape(E, 1, H),
               W2.astype(jnp.bfloat16))
    return (out.reshape(orig_shape), bias)
